# Initial kernel scaffold; baseline (speedup 1.0000x reference)
import jax
import jax.numpy as jnp
from jax import lax
from jax.experimental import pallas as pl
from jax.experimental.pallas import tpu as pltpu
from jax.experimental.pallas import tpu_sc as plsc

N = 10000
E = 320000
D_IN = 128
EMB = 16
HID = 128
NG = 64
MAX_EMBS = 30
CPW = 16

NC = 2
NS = 16
NW = NC * NS
CH = 80
EDGES_PER_TILE = E // NW
CHUNKS_PER_TILE = EDGES_PER_TILE // CH
ROWS_PER_SUB = N // NS

BLK = 2560
F32 = jnp.float32


def _silu(v):
    return v * jax.nn.sigmoid(v)


def _dot(a, b):
    return jnp.dot(a, b, preferred_element_type=F32)


def _pre_body(x_ref, ids_ref, idt_ref, win_ref, bin_ref, w1r_ref, w1c_ref,
              b1_ref, h_ref, hr_ref, hc_ref):
    ids = ids_ref[...]
    onehot = (ids == lax.broadcasted_iota(jnp.int32, (1, MAX_EMBS), 1)).astype(F32)
    emb = _dot(onehot, idt_ref[...])
    w = win_ref[...]
    h = _dot(x_ref[...], w[:D_IN]) + _dot(emb, w[D_IN:]) + bin_ref[...]
    h_ref[...] = h
    hr_ref[...] = _dot(h, w1r_ref[...]) + b1_ref[...]
    hc_ref[...] = _dot(h, w1c_ref[...])


_pre_call = pl.pallas_call(
    _pre_body,
    out_shape=[jax.ShapeDtypeStruct((N, HID), F32)] * 3,
)


def _edge_math(hr_ref, hc_ref, cr_ref, cc_ref, ea_ref, wrad_ref, wea_ref,
               w2_ref, b2_ref):
    diff = cr_ref[...] - cc_ref[...]
    radial = jnp.sum(diff * diff, axis=1, keepdims=True)
    z1 = (hr_ref[...] + hc_ref[...] + radial * wrad_ref[...]
          + ea_ref[...] * wea_ref[...])
    m = _silu(_dot(_silu(z1), w2_ref[...]) + b2_ref[...])
    return m, diff


def _edge_body(hr_ref, hc_ref, cr_ref, cc_ref, ea_ref, wrad_ref, wea_ref,
               w2_ref, b2_ref, wc1_ref, bc1_ref, wc2_ref, m_ref, tr_ref):
    m, diff = _edge_math(hr_ref, hc_ref, cr_ref, cc_ref, ea_ref, wrad_ref,
                         wea_ref, w2_ref, b2_ref)
    m_ref[...] = m
    u = _silu(_dot(m, wc1_ref[...]) + bc1_ref[...])
    cm = _dot(u, wc2_ref[...])
    tr = diff * cm
    lane = lax.broadcasted_iota(jnp.int32, (1, CPW), 1)
    tr_ref[...] = jnp.where(lane == 3, 1.0, tr)


def _edge_body_nc(hr_ref, hc_ref, cr_ref, cc_ref, ea_ref, wrad_ref, wea_ref,
                  w2_ref, b2_ref, m_ref):
    m, _ = _edge_math(hr_ref, hc_ref, cr_ref, cc_ref, ea_ref, wrad_ref,
                      wea_ref, w2_ref, b2_ref)
    m_ref[...] = m


_edge_in_specs = [
    pl.BlockSpec((BLK, HID), lambda i: (i, 0)),
    pl.BlockSpec((BLK, HID), lambda i: (i, 0)),
    pl.BlockSpec((BLK, CPW), lambda i: (i, 0)),
    pl.BlockSpec((BLK, CPW), lambda i: (i, 0)),
    pl.BlockSpec((BLK, 1), lambda i: (i, 0)),
    pl.BlockSpec((1, HID), lambda i: (0, 0)),
    pl.BlockSpec((1, HID), lambda i: (0, 0)),
    pl.BlockSpec((HID, HID), lambda i: (0, 0)),
    pl.BlockSpec((1, HID), lambda i: (0, 0)),
    pl.BlockSpec((HID, HID), lambda i: (0, 0)),
    pl.BlockSpec((1, HID), lambda i: (0, 0)),
    pl.BlockSpec((HID, 1), lambda i: (0, 0)),
]

_edge_call = pl.pallas_call(
    _edge_body,
    grid=(E // BLK,),
    in_specs=_edge_in_specs,
    out_specs=[pl.BlockSpec((BLK, HID), lambda i: (i, 0)),
               pl.BlockSpec((BLK, CPW), lambda i: (i, 0))],
    out_shape=[jax.ShapeDtypeStruct((E, HID), F32),
               jax.ShapeDtypeStruct((E, CPW), F32)],
)

_edge_nc_call = pl.pallas_call(
    _edge_body_nc,
    grid=(E // BLK,),
    in_specs=_edge_in_specs[:9],
    out_specs=pl.BlockSpec((BLK, HID), lambda i: (i, 0)),
    out_shape=jax.ShapeDtypeStruct((E, HID), F32),
)


def _node_body(h_ref, c_ref, p1_ref, p2_ref, wn1h_ref, wn1a_ref, bn1_ref,
               wn2_ref, bn2_ref, w1r_ref, w1c_ref, b1_ref,
               h_o, c_o, hr_o, hc_o):
    agg = p1_ref[0] + p1_ref[1]
    t16 = p2_ref[0] + p2_ref[1]
    cnt = jnp.clip(t16[:, 3:4], 1.0, None)
    lane = lax.broadcasted_iota(jnp.int32, (1, CPW), 1)
    c_o[...] = c_ref[...] + jnp.where(lane < 3, t16, 0.0) / cnt
    h = h_ref[...]
    z = _dot(h, wn1h_ref[...]) + _dot(agg, wn1a_ref[...]) + bn1_ref[...]
    hn = h + _dot(_silu(z), wn2_ref[...]) + bn2_ref[...]
    h_o[...] = hn
    hr_o[...] = _dot(hn, w1r_ref[...]) + b1_ref[...]
    hc_o[...] = _dot(hn, w1c_ref[...])


_node_call = pl.pallas_call(
    _node_body,
    out_shape=[jax.ShapeDtypeStruct((N, HID), F32),
               jax.ShapeDtypeStruct((N, CPW), F32),
               jax.ShapeDtypeStruct((N, HID), F32),
               jax.ShapeDtypeStruct((N, HID), F32)],
)


def _node_final_body(h_ref, p1_ref, batch_ref, wn1h_ref, wn1a_ref, bn1_ref,
                     wn2_ref, bn2_ref, wout_ref, bout_ref, wfc_ref, bfc_ref,
                     o_ref, hscr):
    agg = p1_ref[0] + p1_ref[1]
    h = h_ref[...]
    z = _dot(h, wn1h_ref[...]) + _dot(agg, wn1a_ref[...]) + bn1_ref[...]
    hn = h + _dot(_silu(z), wn2_ref[...]) + bn2_ref[...]
    hscr[...] = _dot(hn, wout_ref[...]) + bout_ref[...]
    b = batch_ref[...]
    rowid = lax.broadcasted_iota(jnp.int32, (NG, 1), 0)
    neg = jnp.full((NG, HID), -jnp.inf, F32)

    def body(g, acc):
        mg = jnp.where(b == g, hscr[...], -jnp.inf)
        rmax = jnp.max(mg, axis=0, keepdims=True)
        return jnp.where(rowid == g, rmax, acc)

    gmat = lax.fori_loop(0, NG, body, neg)
    o_ref[...] = _dot(gmat, wfc_ref[...]) + bfc_ref[...]


_node_final_call = pl.pallas_call(
    _node_final_body,
    out_shape=jax.ShapeDtypeStruct((NG, HID), F32),
    scratch_shapes=[pltpu.VMEM((N, HID), F32)],
)


_sc_mesh = plsc.VectorSubcoreMesh(core_axis_name="c", subcore_axis_name="s")


def _gather_body(hr_hbm, hc_hbm, cp_hbm, row_hbm, col_hbm,
                 ohr, ohc, ocr, occ,
                 ridx, cidx, bhr, bhc, bcr, bcc, sem):
    cid = lax.axis_index("c")
    sid = lax.axis_index("s")
    wid = sid * NC + cid
    tile_base = wid * EDGES_PER_TILE

    @pl.loop(0, CHUNKS_PER_TILE)
    def _(i):
        base = tile_base + i * CH
        pltpu.sync_copy(row_hbm.at[pl.ds(base, CH)], ridx)
        pltpu.sync_copy(col_hbm.at[pl.ds(base, CH)], cidx)
        g1 = pltpu.async_copy(hr_hbm.at[ridx], bhr, sem)
        g2 = pltpu.async_copy(hc_hbm.at[cidx], bhc, sem)
        g3 = pltpu.async_copy(cp_hbm.at[ridx], bcr, sem)
        g4 = pltpu.async_copy(cp_hbm.at[cidx], bcc, sem)
        g1.wait()
        g2.wait()
        g3.wait()
        g4.wait()
        w1 = pltpu.async_copy(bhr, ohr.at[pl.ds(base, CH)], sem)
        w2 = pltpu.async_copy(bhc, ohc.at[pl.ds(base, CH)], sem)
        w3 = pltpu.async_copy(bcr, ocr.at[pl.ds(base, CH)], sem)
        w4 = pltpu.async_copy(bcc, occ.at[pl.ds(base, CH)], sem)
        w1.wait()
        w2.wait()
        w3.wait()
        w4.wait()


def _sc_gather(hr, hc, cp, row, col):
    f = pl.kernel(
        _gather_body,
        out_type=[jax.ShapeDtypeStruct((E, HID), F32),
                  jax.ShapeDtypeStruct((E, HID), F32),
                  jax.ShapeDtypeStruct((E, CPW), F32),
                  jax.ShapeDtypeStruct((E, CPW), F32)],
        mesh=_sc_mesh,
        scratch_types=[pltpu.VMEM((CH,), jnp.int32),
                       pltpu.VMEM((CH,), jnp.int32),
                       pltpu.VMEM((CH, HID), F32),
                       pltpu.VMEM((CH, HID), F32),
                       pltpu.VMEM((CH, CPW), F32),
                       pltpu.VMEM((CH, CPW), F32),
                       pltpu.SemaphoreType.DMA],
    )
    return f(hr, hc, cp, row, col)


def _scatter_body(m_hbm, tr_hbm, row_hbm, z1_hbm, z2_hbm, o1_hbm, o2_hbm,
                  idxv, mbuf, tbuf, acc1, acc2, sem):
    cid = lax.axis_index("c")
    sid = lax.axis_index("s")
    wid = sid * NC + cid
    tile_base = wid * EDGES_PER_TILE
    r0 = sid * ROWS_PER_SUB
    pltpu.sync_copy(z1_hbm.at[pl.ds(r0, ROWS_PER_SUB)],
                    acc1.at[pl.ds(r0, ROWS_PER_SUB)])
    pltpu.sync_copy(z2_hbm.at[pl.ds(r0, ROWS_PER_SUB)],
                    acc2.at[pl.ds(r0, ROWS_PER_SUB)])
    plsc.subcore_barrier()

    @pl.loop(0, CHUNKS_PER_TILE)
    def _(i):
        base = tile_base + i * CH
        pltpu.sync_copy(row_hbm.at[pl.ds(base, CH)], idxv)
        g1 = pltpu.async_copy(m_hbm.at[pl.ds(base, CH)], mbuf, sem)
        g2 = pltpu.async_copy(tr_hbm.at[pl.ds(base, CH)], tbuf, sem)
        g1.wait()
        g2.wait()
        pltpu.sync_copy(mbuf, acc1.at[idxv], add=True)
        pltpu.sync_copy(tbuf, acc2.at[idxv], add=True)

    plsc.subcore_barrier()
    pltpu.sync_copy(acc1.at[pl.ds(r0, ROWS_PER_SUB)],
                    o1_hbm.at[cid, pl.ds(r0, ROWS_PER_SUB)])
    pltpu.sync_copy(acc2.at[pl.ds(r0, ROWS_PER_SUB)],
                    o2_hbm.at[cid, pl.ds(r0, ROWS_PER_SUB)])


def _sc_scatter(m_e, tr_e, row, z1, z2):
    f = pl.kernel(
        _scatter_body,
        out_type=[jax.ShapeDtypeStruct((NC, N, HID), F32),
                  jax.ShapeDtypeStruct((NC, N, CPW), F32)],
        mesh=_sc_mesh,
        scratch_types=[pltpu.VMEM((CH,), jnp.int32),
                       pltpu.VMEM((CH, HID), F32),
                       pltpu.VMEM((CH, CPW), F32),
                       pltpu.VMEM_SHARED((N, HID), F32),
                       pltpu.VMEM_SHARED((N, CPW), F32),
                       pltpu.SemaphoreType.DMA],
    )
    return f(m_e, tr_e, row, z1, z2)


def _scatter1_body(m_hbm, row_hbm, z1_hbm, o1_hbm, idxv, mbuf, acc1, sem):
    cid = lax.axis_index("c")
    sid = lax.axis_index("s")
    wid = sid * NC + cid
    tile_base = wid * EDGES_PER_TILE
    r0 = sid * ROWS_PER_SUB
    pltpu.sync_copy(z1_hbm.at[pl.ds(r0, ROWS_PER_SUB)],
                    acc1.at[pl.ds(r0, ROWS_PER_SUB)])
    plsc.subcore_barrier()

    @pl.loop(0, CHUNKS_PER_TILE)
    def _(i):
        base = tile_base + i * CH
        pltpu.sync_copy(row_hbm.at[pl.ds(base, CH)], idxv)
        pltpu.sync_copy(m_hbm.at[pl.ds(base, CH)], mbuf)
        pltpu.sync_copy(mbuf, acc1.at[idxv], add=True)

    plsc.subcore_barrier()
    pltpu.sync_copy(acc1.at[pl.ds(r0, ROWS_PER_SUB)],
                    o1_hbm.at[cid, pl.ds(r0, ROWS_PER_SUB)])


def _sc_scatter1(m_e, row, z1):
    f = pl.kernel(
        _scatter1_body,
        out_type=jax.ShapeDtypeStruct((NC, N, HID), F32),
        mesh=_sc_mesh,
        scratch_types=[pltpu.VMEM((CH,), jnp.int32),
                       pltpu.VMEM((CH, HID), F32),
                       pltpu.VMEM_SHARED((N, HID), F32),
                       pltpu.SemaphoreType.DMA],
    )
    return f(m_e, row, z1)


def kernel(x, edge_index, coord, edge_attr, batch, ids, id_table, W_in, b_in,
           We1, be1, We2, be2, Wn1, bn1, Wn2, bn2, Wc1, bc1, Wc2,
           W_out, b_out, W_fc, b_fc):
    row = edge_index[0].astype(jnp.int32)
    col = edge_index[1].astype(jnp.int32)
    cpad = jnp.pad(coord.astype(F32), ((0, 0), (0, CPW - 3)))
    ids2 = ids.reshape(N, 1).astype(jnp.int32)
    batch2 = batch.reshape(N, 1).astype(jnp.int32)
    z1 = jnp.zeros((N, HID), F32)
    z2 = jnp.zeros((N, CPW), F32)

    h, hr, hc = _pre_call(x, ids2, id_table, W_in, b_in.reshape(1, HID),
                          We1[0, :HID], We1[0, HID:2 * HID],
                          be1[0].reshape(1, HID))
    c = cpad
    out = None
    for i in range(3):
        hr_g, hc_g, cr_g, cc_g = _sc_gather(hr, hc, c, row, col)
        wrad = We1[i, 2 * HID:2 * HID + 1]
        wea = We1[i, 2 * HID + 1:2 * HID + 2]
        if i < 2:
            m_e, tr_e = _edge_call(hr_g, hc_g, cr_g, cc_g, edge_attr, wrad,
                                   wea, We2[i], be2[i].reshape(1, HID),
                                   Wc1[i], bc1[i].reshape(1, HID), Wc2[i])
            p1, p2 = _sc_scatter(m_e, tr_e, row, z1, z2)
            h, c, hr, hc = _node_call(h, c, p1, p2, Wn1[i, :HID],
                                      Wn1[i, HID:], bn1[i].reshape(1, HID),
                                      Wn2[i], bn2[i].reshape(1, HID),
                                      We1[i + 1, :HID],
                                      We1[i + 1, HID:2 * HID],
                                      be1[i + 1].reshape(1, HID))
        else:
            m_e = _edge_nc_call(hr_g, hc_g, cr_g, cc_g, edge_attr, wrad, wea,
                                We2[i], be2[i].reshape(1, HID))
            p1 = _sc_scatter1(m_e, row, z1)
            out = _node_final_call(h, p1, batch2, Wn1[i, :HID], Wn1[i, HID:],
                                   bn1[i].reshape(1, HID), Wn2[i],
                                   bn2[i].reshape(1, HID), W_out,
                                   b_out.reshape(1, HID), W_fc,
                                   b_fc.reshape(1, HID))
    return out


# speedup vs baseline: 2.3478x; 2.3478x over previous
import jax
import jax.numpy as jnp
from jax import lax
from jax.experimental import pallas as pl
from jax.experimental.pallas import tpu as pltpu
from jax.experimental.pallas import tpu_sc as plsc

N = 10000
E = 320000
D_IN = 128
EMB = 16
HID = 128
NG = 64
MAX_EMBS = 30
CPW = 16
CTW = 128

NC = 2
NS = 16
NW = NC * NS
CH = 80
EDGES_PER_TILE = E // NW
CHUNKS_PER_TILE = EDGES_PER_TILE // CH
NPAD = 10240
ROWS_PER_SUB = NPAD // NS

BLK = 2560
F32 = jnp.float32


def _silu(v):
    return v * jax.nn.sigmoid(v)


def _dot(a, b):
    return jnp.dot(a, b, preferred_element_type=F32)


def _pre_body(x_ref, ids_ref, idt_ref, win_ref, bin_ref, w1r_ref, w1c_ref,
              b1_ref, h_ref, hr_ref, hc_ref):
    ids = ids_ref[...]
    onehot = (ids == lax.broadcasted_iota(jnp.int32, (1, MAX_EMBS), 1)).astype(F32)
    emb = _dot(onehot, idt_ref[...])
    w = win_ref[...]
    h = _dot(x_ref[...], w[:D_IN]) + _dot(emb, w[D_IN:]) + bin_ref[...]
    h_ref[...] = h
    hr_ref[...] = _dot(h, w1r_ref[...]) + b1_ref[...]
    hc_ref[...] = _dot(h, w1c_ref[...])


_pre_call = pl.pallas_call(
    _pre_body,
    out_shape=[jax.ShapeDtypeStruct((N, HID), F32)] * 3,
)


def _edge_math(hr_ref, hc_ref, cr_ref, cc_ref, ea_ref, wrad_ref, wea_ref,
               w2_ref, b2_ref):
    diff = cr_ref[...] - cc_ref[...]
    radial = jnp.sum(diff * diff, axis=1, keepdims=True)
    z1 = (hr_ref[...] + hc_ref[...] + radial * wrad_ref[...]
          + ea_ref[...] * wea_ref[...])
    m = _silu(_dot(_silu(z1), w2_ref[...]) + b2_ref[...])
    return m, diff


def _edge_body(hr_ref, hc_ref, cr_ref, cc_ref, ea_ref, wrad_ref, wea_ref,
               w2_ref, b2_ref, wc1_ref, bc1_ref, wc2_ref, m_ref, tr_ref):
    m, diff = _edge_math(hr_ref, hc_ref, cr_ref, cc_ref, ea_ref, wrad_ref,
                         wea_ref, w2_ref, b2_ref)
    m_ref[...] = m
    u = _silu(_dot(m, wc1_ref[...]) + bc1_ref[...])
    cm = _dot(u, wc2_ref[...])
    tr = diff * cm
    lane = lax.broadcasted_iota(jnp.int32, (1, CTW), 1)
    tr_ref[...] = jnp.where(lane == 3, 1.0, tr)


def _edge_body_nc(hr_ref, hc_ref, cr_ref, cc_ref, ea_ref, wrad_ref, wea_ref,
                  w2_ref, b2_ref, m_ref):
    m, _ = _edge_math(hr_ref, hc_ref, cr_ref, cc_ref, ea_ref, wrad_ref,
                      wea_ref, w2_ref, b2_ref)
    m_ref[...] = m


_edge_in_specs = [
    pl.BlockSpec((BLK, HID), lambda i: (i, 0)),
    pl.BlockSpec((BLK, HID), lambda i: (i, 0)),
    pl.BlockSpec((BLK, CTW), lambda i: (i, 0)),
    pl.BlockSpec((BLK, CTW), lambda i: (i, 0)),
    pl.BlockSpec((BLK, 1), lambda i: (i, 0)),
    pl.BlockSpec((1, HID), lambda i: (0, 0)),
    pl.BlockSpec((1, HID), lambda i: (0, 0)),
    pl.BlockSpec((HID, HID), lambda i: (0, 0)),
    pl.BlockSpec((1, HID), lambda i: (0, 0)),
    pl.BlockSpec((HID, HID), lambda i: (0, 0)),
    pl.BlockSpec((1, HID), lambda i: (0, 0)),
    pl.BlockSpec((HID, 1), lambda i: (0, 0)),
]

_edge_call = pl.pallas_call(
    _edge_body,
    grid=(E // BLK,),
    in_specs=_edge_in_specs,
    out_specs=[pl.BlockSpec((BLK, HID), lambda i: (i, 0)),
               pl.BlockSpec((BLK, CTW), lambda i: (i, 0))],
    out_shape=[jax.ShapeDtypeStruct((E, HID), F32),
               jax.ShapeDtypeStruct((E, CTW), F32)],
)

_edge_nc_call = pl.pallas_call(
    _edge_body_nc,
    grid=(E // BLK,),
    in_specs=_edge_in_specs[:9],
    out_specs=pl.BlockSpec((BLK, HID), lambda i: (i, 0)),
    out_shape=jax.ShapeDtypeStruct((E, HID), F32),
)


def _node_body(h_ref, c_ref, p1_ref, p2_ref, wn1h_ref, wn1a_ref, bn1_ref,
               wn2_ref, bn2_ref, w1r_ref, w1c_ref, b1_ref,
               h_o, c_o, hr_o, hc_o):
    agg = (p1_ref[0] + p1_ref[1])[:N]
    t16 = (p2_ref[0] + p2_ref[1])[:N]
    cnt = jnp.clip(t16[:, 3:4], 1.0, None)
    lane = lax.broadcasted_iota(jnp.int32, (1, CTW), 1)
    c_o[...] = c_ref[...] + jnp.where(lane < 3, t16, 0.0) / cnt
    h = h_ref[...]
    z = _dot(h, wn1h_ref[...]) + _dot(agg, wn1a_ref[...]) + bn1_ref[...]
    hn = h + _dot(_silu(z), wn2_ref[...]) + bn2_ref[...]
    h_o[...] = hn
    hr_o[...] = _dot(hn, w1r_ref[...]) + b1_ref[...]
    hc_o[...] = _dot(hn, w1c_ref[...])


_node_call = pl.pallas_call(
    _node_body,
    out_shape=[jax.ShapeDtypeStruct((N, HID), F32),
               jax.ShapeDtypeStruct((N, CTW), F32),
               jax.ShapeDtypeStruct((N, HID), F32),
               jax.ShapeDtypeStruct((N, HID), F32)],
)


def _node_final_body(h_ref, p1_ref, batch_ref, wn1h_ref, wn1a_ref, bn1_ref,
                     wn2_ref, bn2_ref, wout_ref, bout_ref, wfc_ref, bfc_ref,
                     o_ref, hscr):
    agg = (p1_ref[0] + p1_ref[1])[:N]
    h = h_ref[...]
    z = _dot(h, wn1h_ref[...]) + _dot(agg, wn1a_ref[...]) + bn1_ref[...]
    hn = h + _dot(_silu(z), wn2_ref[...]) + bn2_ref[...]
    hscr[...] = _dot(hn, wout_ref[...]) + bout_ref[...]
    b = batch_ref[...]
    rowid = lax.broadcasted_iota(jnp.int32, (NG, 1), 0)
    neg = jnp.full((NG, HID), -jnp.inf, F32)

    def body(g, acc):
        mg = jnp.where(b == g, hscr[...], -jnp.inf)
        rmax = jnp.max(mg, axis=0, keepdims=True)
        return jnp.where(rowid == g, rmax, acc)

    gmat = lax.fori_loop(0, NG, body, neg)
    o_ref[...] = _dot(gmat, wfc_ref[...]) + bfc_ref[...]


_node_final_call = pl.pallas_call(
    _node_final_body,
    out_shape=jax.ShapeDtypeStruct((NG, HID), F32),
    scratch_shapes=[pltpu.VMEM((N, HID), F32)],
)


_sc_mesh = plsc.VectorSubcoreMesh(core_axis_name="c", subcore_axis_name="s")


def _gather_body(hr_hbm, hc_hbm, cp_hbm, row_hbm, col_hbm,
                 ohr, ohc, ocr, occ,
                 ridx, cidx, bhr, bhc, bcr, bcc, sem):
    cid = lax.axis_index("c")
    sid = lax.axis_index("s")
    wid = sid * NC + cid
    tile_base = wid * EDGES_PER_TILE

    @pl.loop(0, CHUNKS_PER_TILE)
    def _(i):
        base = tile_base + i * CH
        pltpu.sync_copy(row_hbm.at[pl.ds(base, CH)], ridx)
        pltpu.sync_copy(col_hbm.at[pl.ds(base, CH)], cidx)
        pltpu.sync_copy(hr_hbm.at[ridx], bhr)
        pltpu.sync_copy(hc_hbm.at[cidx], bhc)
        pltpu.sync_copy(cp_hbm.at[ridx], bcr)
        pltpu.sync_copy(cp_hbm.at[cidx], bcc)
        pltpu.sync_copy(bhr, ohr.at[pl.ds(base, CH)])
        pltpu.sync_copy(bhc, ohc.at[pl.ds(base, CH)])
        pltpu.sync_copy(bcr, ocr.at[pl.ds(base, CH)])
        pltpu.sync_copy(bcc, occ.at[pl.ds(base, CH)])


def _sc_gather(hr, hc, cp, row, col):
    f = pl.kernel(
        _gather_body,
        out_type=[jax.ShapeDtypeStruct((E, HID), F32),
                  jax.ShapeDtypeStruct((E, HID), F32),
                  jax.ShapeDtypeStruct((E, CTW), F32),
                  jax.ShapeDtypeStruct((E, CTW), F32)],
        mesh=_sc_mesh,
        scratch_types=[pltpu.VMEM((CH,), jnp.int32),
                       pltpu.VMEM((CH,), jnp.int32),
                       pltpu.VMEM((CH, HID), F32),
                       pltpu.VMEM((CH, HID), F32),
                       pltpu.VMEM((CH, CTW), F32),
                       pltpu.VMEM((CH, CTW), F32),
                       pltpu.SemaphoreType.DMA],
    )
    return f(hr, hc, cp, row, col)


def _scatter1_body(m_hbm, row_hbm, z1_hbm, o1_hbm, idxv, mbuf, acc1, sem):
    cid = lax.axis_index("c")
    sid = lax.axis_index("s")
    wid = sid * NC + cid
    tile_base = wid * EDGES_PER_TILE
    r0 = sid * ROWS_PER_SUB
    pltpu.sync_copy(z1_hbm.at[pl.ds(r0, ROWS_PER_SUB)],
                    acc1.at[pl.ds(r0, ROWS_PER_SUB)])
    plsc.subcore_barrier()

    @pl.loop(0, CHUNKS_PER_TILE)
    def _(i):
        base = tile_base + i * CH
        pltpu.sync_copy(row_hbm.at[pl.ds(base, CH)], idxv)
        pltpu.sync_copy(m_hbm.at[pl.ds(base, CH)], mbuf)
        pltpu.sync_copy(mbuf, acc1.at[idxv], add=True)

    plsc.subcore_barrier()
    pltpu.sync_copy(acc1.at[pl.ds(r0, ROWS_PER_SUB)],
                    o1_hbm.at[cid, pl.ds(r0, ROWS_PER_SUB)])


def _sc_scatter1(m_e, row, z1):
    f = pl.kernel(
        _scatter1_body,
        out_type=jax.ShapeDtypeStruct((NC, NPAD, HID), F32),
        mesh=_sc_mesh,
        scratch_types=[pltpu.VMEM((CH,), jnp.int32),
                       pltpu.VMEM((CH, HID), F32),
                       pltpu.VMEM_SHARED((NPAD, HID), F32),
                       pltpu.SemaphoreType.DMA],
    )
    return f(m_e, row, z1)


def kernel(x, edge_index, coord, edge_attr, batch, ids, id_table, W_in, b_in,
           We1, be1, We2, be2, Wn1, bn1, Wn2, bn2, Wc1, bc1, Wc2,
           W_out, b_out, W_fc, b_fc):
    row = edge_index[0].astype(jnp.int32)
    col = edge_index[1].astype(jnp.int32)
    cpad = jnp.pad(coord.astype(F32), ((0, 0), (0, CTW - 3)))
    ids2 = ids.reshape(N, 1).astype(jnp.int32)
    batch2 = batch.reshape(N, 1).astype(jnp.int32)
    z1 = jnp.zeros((NPAD, HID), F32)

    h, hr, hc = _pre_call(x, ids2, id_table, W_in, b_in.reshape(1, HID),
                          We1[0, :HID], We1[0, HID:2 * HID],
                          be1[0].reshape(1, HID))
    c = cpad
    out = None
    for i in range(3):
        hr_g, hc_g, cr_g, cc_g = _sc_gather(hr, hc, c, row, col)
        wrad = We1[i, 2 * HID:2 * HID + 1]
        wea = We1[i, 2 * HID + 1:2 * HID + 2]
        if i < 2:
            m_e, tr_e = _edge_call(hr_g, hc_g, cr_g, cc_g, edge_attr, wrad,
                                   wea, We2[i], be2[i].reshape(1, HID),
                                   Wc1[i], bc1[i].reshape(1, HID), Wc2[i])
            p1 = _sc_scatter1(m_e, row, z1)
            p2 = _sc_scatter1(tr_e, row, z1)
            h, c, hr, hc = _node_call(h, c, p1, p2, Wn1[i, :HID],
                                      Wn1[i, HID:], bn1[i].reshape(1, HID),
                                      Wn2[i], bn2[i].reshape(1, HID),
                                      We1[i + 1, :HID],
                                      We1[i + 1, HID:2 * HID],
                                      be1[i + 1].reshape(1, HID))
        else:
            m_e = _edge_nc_call(hr_g, hc_g, cr_g, cc_g, edge_attr, wrad, wea,
                                We2[i], be2[i].reshape(1, HID))
            p1 = _sc_scatter1(m_e, row, z1)
            out = _node_final_call(h, p1, batch2, Wn1[i, :HID], Wn1[i, HID:],
                                   bn1[i].reshape(1, HID), Wn2[i],
                                   bn2[i].reshape(1, HID), W_out,
                                   b_out.reshape(1, HID), W_fc,
                                   b_fc.reshape(1, HID))
    return out


# speedup vs baseline: 3.2113x; 1.3678x over previous
import jax
import jax.numpy as jnp
from jax import lax
from jax.experimental import pallas as pl
from jax.experimental.pallas import tpu as pltpu
from jax.experimental.pallas import tpu_sc as plsc

N = 10000
E = 320000
D_IN = 128
EMB = 16
HID = 128
NG = 64
MAX_EMBS = 30
CPW = 16
CTW = 128

NC = 2
NS = 16
NW = NC * NS
CH = 80
EDGES_PER_TILE = E // NW
CHUNKS_PER_TILE = EDGES_PER_TILE // CH
NPAD = 10240
ROWS_PER_SUB = NPAD // NS

BLK = 2560
F32 = jnp.float32


def _silu(v):
    return v * jax.nn.sigmoid(v)


def _dot(a, b):
    return jnp.dot(a, b, preferred_element_type=F32)


def _pre_body(x_ref, ids_ref, idt_ref, win_ref, bin_ref, w1r_ref, w1c_ref,
              b1_ref, h_ref, hr_ref, hc_ref):
    ids = ids_ref[...]
    onehot = (ids == lax.broadcasted_iota(jnp.int32, (1, MAX_EMBS), 1)).astype(F32)
    emb = _dot(onehot, idt_ref[...])
    w = win_ref[...]
    h = _dot(x_ref[...], w[:D_IN]) + _dot(emb, w[D_IN:]) + bin_ref[...]
    h_ref[...] = h
    hr_ref[...] = _dot(h, w1r_ref[...]) + b1_ref[...]
    hc_ref[...] = _dot(h, w1c_ref[...])


_pre_call = pl.pallas_call(
    _pre_body,
    out_shape=[jax.ShapeDtypeStruct((N, HID), F32)] * 3,
)


def _edge_math(hr_ref, hc_ref, cr_ref, cc_ref, ea_ref, wrad_ref, wea_ref,
               w2_ref, b2_ref):
    diff = cr_ref[...] - cc_ref[...]
    radial = jnp.sum(diff * diff, axis=1, keepdims=True)
    z1 = (hr_ref[...] + hc_ref[...] + radial * wrad_ref[...]
          + ea_ref[...] * wea_ref[...])
    m = _silu(_dot(_silu(z1), w2_ref[...]) + b2_ref[...])
    return m, diff


def _edge_body(hr_ref, hc_ref, cr_ref, cc_ref, ea_ref, wrad_ref, wea_ref,
               w2_ref, b2_ref, wc1_ref, bc1_ref, wc2_ref, m_ref, tr_ref):
    m, diff = _edge_math(hr_ref, hc_ref, cr_ref, cc_ref, ea_ref, wrad_ref,
                         wea_ref, w2_ref, b2_ref)
    m_ref[...] = m
    u = _silu(_dot(m, wc1_ref[...]) + bc1_ref[...])
    cm = _dot(u, wc2_ref[...])
    tr = diff * cm
    lane = lax.broadcasted_iota(jnp.int32, (1, CTW), 1)
    tr_ref[...] = jnp.where(lane == 3, 1.0, tr)


def _edge_body_nc(hr_ref, hc_ref, cr_ref, cc_ref, ea_ref, wrad_ref, wea_ref,
                  w2_ref, b2_ref, m_ref):
    m, _ = _edge_math(hr_ref, hc_ref, cr_ref, cc_ref, ea_ref, wrad_ref,
                      wea_ref, w2_ref, b2_ref)
    m_ref[...] = m


_edge_in_specs = [
    pl.BlockSpec((BLK, HID), lambda i: (i, 0)),
    pl.BlockSpec((BLK, HID), lambda i: (i, 0)),
    pl.BlockSpec((BLK, CTW), lambda i: (i, 0)),
    pl.BlockSpec((BLK, CTW), lambda i: (i, 0)),
    pl.BlockSpec((BLK, 1), lambda i: (i, 0)),
    pl.BlockSpec((1, HID), lambda i: (0, 0)),
    pl.BlockSpec((1, HID), lambda i: (0, 0)),
    pl.BlockSpec((HID, HID), lambda i: (0, 0)),
    pl.BlockSpec((1, HID), lambda i: (0, 0)),
    pl.BlockSpec((HID, HID), lambda i: (0, 0)),
    pl.BlockSpec((1, HID), lambda i: (0, 0)),
    pl.BlockSpec((HID, 1), lambda i: (0, 0)),
]

_edge_call = pl.pallas_call(
    _edge_body,
    grid=(E // BLK,),
    in_specs=_edge_in_specs,
    out_specs=[pl.BlockSpec((BLK, HID), lambda i: (i, 0)),
               pl.BlockSpec((BLK, CTW), lambda i: (i, 0))],
    out_shape=[jax.ShapeDtypeStruct((E, HID), F32),
               jax.ShapeDtypeStruct((E, CTW), F32)],
)

_edge_nc_call = pl.pallas_call(
    _edge_body_nc,
    grid=(E // BLK,),
    in_specs=_edge_in_specs[:9],
    out_specs=pl.BlockSpec((BLK, HID), lambda i: (i, 0)),
    out_shape=jax.ShapeDtypeStruct((E, HID), F32),
)


def _node_body(h_ref, c_ref, p1_ref, p2_ref, wn1h_ref, wn1a_ref, bn1_ref,
               wn2_ref, bn2_ref, w1r_ref, w1c_ref, b1_ref,
               h_o, c_o, hr_o, hc_o):
    agg = (p1_ref[0] + p1_ref[1])[:N]
    t16 = (p2_ref[0] + p2_ref[1])[:N]
    cnt = jnp.clip(t16[:, 3:4], 1.0, None)
    lane = lax.broadcasted_iota(jnp.int32, (1, CTW), 1)
    c_o[...] = c_ref[...] + jnp.where(lane < 3, t16, 0.0) / cnt
    h = h_ref[...]
    z = _dot(h, wn1h_ref[...]) + _dot(agg, wn1a_ref[...]) + bn1_ref[...]
    hn = h + _dot(_silu(z), wn2_ref[...]) + bn2_ref[...]
    h_o[...] = hn
    hr_o[...] = _dot(hn, w1r_ref[...]) + b1_ref[...]
    hc_o[...] = _dot(hn, w1c_ref[...])


_node_call = pl.pallas_call(
    _node_body,
    out_shape=[jax.ShapeDtypeStruct((N, HID), F32),
               jax.ShapeDtypeStruct((N, CTW), F32),
               jax.ShapeDtypeStruct((N, HID), F32),
               jax.ShapeDtypeStruct((N, HID), F32)],
)


def _node_final_body(h_ref, p1_ref, batch_ref, wn1h_ref, wn1a_ref, bn1_ref,
                     wn2_ref, bn2_ref, wout_ref, bout_ref, wfc_ref, bfc_ref,
                     o_ref, hscr):
    agg = (p1_ref[0] + p1_ref[1])[:N]
    h = h_ref[...]
    z = _dot(h, wn1h_ref[...]) + _dot(agg, wn1a_ref[...]) + bn1_ref[...]
    hn = h + _dot(_silu(z), wn2_ref[...]) + bn2_ref[...]
    hscr[...] = _dot(hn, wout_ref[...]) + bout_ref[...]
    b = batch_ref[...]
    rowid = lax.broadcasted_iota(jnp.int32, (NG, 1), 0)
    neg = jnp.full((NG, HID), -jnp.inf, F32)

    def body(g, acc):
        mg = jnp.where(b == g, hscr[...], -jnp.inf)
        rmax = jnp.max(mg, axis=0, keepdims=True)
        return jnp.where(rowid == g, rmax, acc)

    gmat = lax.fori_loop(0, NG, body, neg)
    o_ref[...] = _dot(gmat, wfc_ref[...]) + bfc_ref[...]


_node_final_call = pl.pallas_call(
    _node_final_body,
    out_shape=jax.ShapeDtypeStruct((NG, HID), F32),
    scratch_shapes=[pltpu.VMEM((N, HID), F32)],
)


_sc_mesh = plsc.VectorSubcoreMesh(core_axis_name="c", subcore_axis_name="s")


def _gather_body(hr_hbm, hc_hbm, cp_hbm, row_hbm, col_hbm,
                 ohr, ohc, ocr, occ,
                 ridx0, cidx0, bhr0, bhc0, bcr0, bcc0,
                 ridx1, cidx1, bhr1, bhc1, bcr1, bcc1,
                 isem0, gsem0, wsem0, isem1, gsem1, wsem1):
    cid = lax.axis_index("c")
    sid = lax.axis_index("s")
    wid = sid * NC + cid
    tile_base = wid * EDGES_PER_TILE
    RIDX = (ridx0, ridx1)
    CIDX = (cidx0, cidx1)
    BHR = (bhr0, bhr1)
    BHC = (bhc0, bhc1)
    BCR = (bcr0, bcr1)
    BCC = (bcc0, bcc1)
    ISEM = (isem0, isem1)
    GSEM = (gsem0, gsem1)
    WSEM = (wsem0, wsem1)

    def issue_idx(b, ch):
        base = tile_base + ch * CH
        pltpu.async_copy(row_hbm.at[pl.ds(base, CH)], RIDX[b], ISEM[b])
        pltpu.async_copy(col_hbm.at[pl.ds(base, CH)], CIDX[b], ISEM[b])

    def wait_idx(b):
        pltpu.make_async_copy(row_hbm.at[pl.ds(tile_base, CH)], RIDX[b],
                              ISEM[b]).wait()
        pltpu.make_async_copy(col_hbm.at[pl.ds(tile_base, CH)], CIDX[b],
                              ISEM[b]).wait()

    def issue_g(b):
        pltpu.async_copy(hr_hbm.at[RIDX[b]], BHR[b], GSEM[b])
        pltpu.async_copy(hc_hbm.at[CIDX[b]], BHC[b], GSEM[b])
        pltpu.async_copy(cp_hbm.at[RIDX[b]], BCR[b], GSEM[b])
        pltpu.async_copy(cp_hbm.at[CIDX[b]], BCC[b], GSEM[b])

    def wait_g(b):
        pltpu.make_async_copy(hr_hbm.at[RIDX[b]], BHR[b], GSEM[b]).wait()
        pltpu.make_async_copy(hc_hbm.at[CIDX[b]], BHC[b], GSEM[b]).wait()
        pltpu.make_async_copy(cp_hbm.at[RIDX[b]], BCR[b], GSEM[b]).wait()
        pltpu.make_async_copy(cp_hbm.at[CIDX[b]], BCC[b], GSEM[b]).wait()

    def issue_wb(b, ch):
        base = tile_base + ch * CH
        pltpu.async_copy(BHR[b], ohr.at[pl.ds(base, CH)], WSEM[b])
        pltpu.async_copy(BHC[b], ohc.at[pl.ds(base, CH)], WSEM[b])
        pltpu.async_copy(BCR[b], ocr.at[pl.ds(base, CH)], WSEM[b])
        pltpu.async_copy(BCC[b], occ.at[pl.ds(base, CH)], WSEM[b])

    def wait_wb(b):
        pltpu.make_async_copy(BHR[b], ohr.at[pl.ds(tile_base, CH)],
                              WSEM[b]).wait()
        pltpu.make_async_copy(BHC[b], ohc.at[pl.ds(tile_base, CH)],
                              WSEM[b]).wait()
        pltpu.make_async_copy(BCR[b], ocr.at[pl.ds(tile_base, CH)],
                              WSEM[b]).wait()
        pltpu.make_async_copy(BCC[b], occ.at[pl.ds(tile_base, CH)],
                              WSEM[b]).wait()

    issue_idx(0, 0)
    wait_idx(0)
    issue_g(0)
    issue_idx(1, 1)

    @pl.loop(0, (CHUNKS_PER_TILE - 1) // 2)
    def _(p):
        a = 2 * p
        bch = 2 * p + 1
        nxt0 = 2 * p + 2
        nxt1 = jnp.minimum(2 * p + 3, CHUNKS_PER_TILE - 1)
        wait_g(0)
        issue_wb(0, a)
        wait_idx(1)
        issue_g(1)
        issue_idx(0, nxt0)
        wait_g(1)
        issue_wb(1, bch)
        issue_idx(1, nxt1)
        wait_wb(0)
        wait_idx(0)
        issue_g(0)
        wait_wb(1)

    wait_g(0)
    issue_wb(0, CHUNKS_PER_TILE - 1)
    wait_wb(0)
    wait_idx(1)


def _sc_gather(hr, hc, cp, row, col):
    bufs = [pltpu.VMEM((CH,), jnp.int32),
            pltpu.VMEM((CH,), jnp.int32),
            pltpu.VMEM((CH, HID), F32),
            pltpu.VMEM((CH, HID), F32),
            pltpu.VMEM((CH, CTW), F32),
            pltpu.VMEM((CH, CTW), F32)]
    f = pl.kernel(
        _gather_body,
        out_type=[jax.ShapeDtypeStruct((E, HID), F32),
                  jax.ShapeDtypeStruct((E, HID), F32),
                  jax.ShapeDtypeStruct((E, CTW), F32),
                  jax.ShapeDtypeStruct((E, CTW), F32)],
        mesh=_sc_mesh,
        scratch_types=bufs + bufs + [pltpu.SemaphoreType.DMA] * 6,
    )
    return f(hr, hc, cp, row, col)


def _scatter1_body(m_hbm, row_hbm, z1_hbm, o1_hbm, idxv, mbuf, acc1, sem):
    cid = lax.axis_index("c")
    sid = lax.axis_index("s")
    wid = sid * NC + cid
    tile_base = wid * EDGES_PER_TILE
    r0 = sid * ROWS_PER_SUB
    pltpu.sync_copy(z1_hbm.at[pl.ds(r0, ROWS_PER_SUB)],
                    acc1.at[pl.ds(r0, ROWS_PER_SUB)])
    plsc.subcore_barrier()

    @pl.loop(0, CHUNKS_PER_TILE)
    def _(i):
        base = tile_base + i * CH
        pltpu.sync_copy(row_hbm.at[pl.ds(base, CH)], idxv)
        pltpu.sync_copy(m_hbm.at[pl.ds(base, CH)], mbuf)
        pltpu.sync_copy(mbuf, acc1.at[idxv], add=True)

    plsc.subcore_barrier()
    pltpu.sync_copy(acc1.at[pl.ds(r0, ROWS_PER_SUB)],
                    o1_hbm.at[cid, pl.ds(r0, ROWS_PER_SUB)])


def _sc_scatter1(m_e, row, z1):
    f = pl.kernel(
        _scatter1_body,
        out_type=jax.ShapeDtypeStruct((NC, NPAD, HID), F32),
        mesh=_sc_mesh,
        scratch_types=[pltpu.VMEM((CH,), jnp.int32),
                       pltpu.VMEM((CH, HID), F32),
                       pltpu.VMEM_SHARED((NPAD, HID), F32),
                       pltpu.SemaphoreType.DMA],
    )
    return f(m_e, row, z1)


def kernel(x, edge_index, coord, edge_attr, batch, ids, id_table, W_in, b_in,
           We1, be1, We2, be2, Wn1, bn1, Wn2, bn2, Wc1, bc1, Wc2,
           W_out, b_out, W_fc, b_fc):
    row = edge_index[0].astype(jnp.int32)
    col = edge_index[1].astype(jnp.int32)
    cpad = jnp.pad(coord.astype(F32), ((0, 0), (0, CTW - 3)))
    ids2 = ids.reshape(N, 1).astype(jnp.int32)
    batch2 = batch.reshape(N, 1).astype(jnp.int32)
    z1 = jnp.zeros((NPAD, HID), F32)

    h, hr, hc = _pre_call(x, ids2, id_table, W_in, b_in.reshape(1, HID),
                          We1[0, :HID], We1[0, HID:2 * HID],
                          be1[0].reshape(1, HID))
    c = cpad
    out = None
    for i in range(3):
        hr_g, hc_g, cr_g, cc_g = _sc_gather(hr, hc, c, row, col)
        wrad = We1[i, 2 * HID:2 * HID + 1]
        wea = We1[i, 2 * HID + 1:2 * HID + 2]
        if i < 2:
            m_e, tr_e = _edge_call(hr_g, hc_g, cr_g, cc_g, edge_attr, wrad,
                                   wea, We2[i], be2[i].reshape(1, HID),
                                   Wc1[i], bc1[i].reshape(1, HID), Wc2[i])
            p1 = _sc_scatter1(m_e, row, z1)
            p2 = _sc_scatter1(tr_e, row, z1)
            h, c, hr, hc = _node_call(h, c, p1, p2, Wn1[i, :HID],
                                      Wn1[i, HID:], bn1[i].reshape(1, HID),
                                      Wn2[i], bn2[i].reshape(1, HID),
                                      We1[i + 1, :HID],
                                      We1[i + 1, HID:2 * HID],
                                      be1[i + 1].reshape(1, HID))
        else:
            m_e = _edge_nc_call(hr_g, hc_g, cr_g, cc_g, edge_attr, wrad, wea,
                                We2[i], be2[i].reshape(1, HID))
            p1 = _sc_scatter1(m_e, row, z1)
            out = _node_final_call(h, p1, batch2, Wn1[i, :HID], Wn1[i, HID:],
                                   bn1[i].reshape(1, HID), Wn2[i],
                                   bn2[i].reshape(1, HID), W_out,
                                   b_out.reshape(1, HID), W_fc,
                                   b_fc.reshape(1, HID))
    return out


# speedup vs baseline: 3.6740x; 1.1441x over previous
import jax
import jax.numpy as jnp
from jax import lax
from jax.experimental import pallas as pl
from jax.experimental.pallas import tpu as pltpu
from jax.experimental.pallas import tpu_sc as plsc

N = 10000
E = 320000
D_IN = 128
EMB = 16
HID = 128
NG = 64
MAX_EMBS = 30
CPW = 16
CTW = 128

NC = 2
NS = 16
NW = NC * NS
CH = 80
EDGES_PER_TILE = E // NW
CHUNKS_PER_TILE = EDGES_PER_TILE // CH
NPAD = 10240
ROWS_PER_SUB = NPAD // NS

BLK = 2560
F32 = jnp.float32


def _silu(v):
    return v * jax.nn.sigmoid(v)


def _dot(a, b):
    return jnp.dot(a, b, preferred_element_type=F32)


def _pre_body(x_ref, ids_ref, idt_ref, win_ref, bin_ref, w1r_ref, w1c_ref,
              b1_ref, h_ref, hr_ref, hc_ref):
    ids = ids_ref[...]
    onehot = (ids == lax.broadcasted_iota(jnp.int32, (1, MAX_EMBS), 1)).astype(F32)
    emb = _dot(onehot, idt_ref[...])
    w = win_ref[...]
    h = _dot(x_ref[...], w[:D_IN]) + _dot(emb, w[D_IN:]) + bin_ref[...]
    h_ref[...] = h
    hr_ref[...] = _dot(h, w1r_ref[...]) + b1_ref[...]
    hc_ref[...] = _dot(h, w1c_ref[...])


_pre_call = pl.pallas_call(
    _pre_body,
    out_shape=[jax.ShapeDtypeStruct((N, HID), F32)] * 3,
)


def _edge_math(hr_ref, hc_ref, cr_ref, cc_ref, ea_ref, wrad_ref, wea_ref,
               w2_ref, b2_ref):
    diff = cr_ref[...] - cc_ref[...]
    radial = jnp.sum(diff * diff, axis=1, keepdims=True)
    z1 = (hr_ref[...] + hc_ref[...] + radial * wrad_ref[...]
          + ea_ref[...] * wea_ref[...])
    m = _silu(_dot(_silu(z1), w2_ref[...]) + b2_ref[...])
    return m, diff


def _edge_body(hr_ref, hc_ref, cr_ref, cc_ref, ea_ref, wrad_ref, wea_ref,
               w2_ref, b2_ref, wc1_ref, bc1_ref, wc2_ref, m_ref, tr_ref):
    m, diff = _edge_math(hr_ref, hc_ref, cr_ref, cc_ref, ea_ref, wrad_ref,
                         wea_ref, w2_ref, b2_ref)
    m_ref[...] = m
    u = _silu(_dot(m, wc1_ref[...]) + bc1_ref[...])
    cm = _dot(u, wc2_ref[...])
    tr = diff * cm
    lane = lax.broadcasted_iota(jnp.int32, (1, CTW), 1)
    tr_ref[...] = jnp.where(lane == 3, 1.0, tr)


def _edge_body_nc(hr_ref, hc_ref, cr_ref, cc_ref, ea_ref, wrad_ref, wea_ref,
                  w2_ref, b2_ref, m_ref):
    m, _ = _edge_math(hr_ref, hc_ref, cr_ref, cc_ref, ea_ref, wrad_ref,
                      wea_ref, w2_ref, b2_ref)
    m_ref[...] = m


_edge_in_specs = [
    pl.BlockSpec((BLK, HID), lambda i: (i, 0)),
    pl.BlockSpec((BLK, HID), lambda i: (i, 0)),
    pl.BlockSpec((BLK, CTW), lambda i: (i, 0)),
    pl.BlockSpec((BLK, CTW), lambda i: (i, 0)),
    pl.BlockSpec((BLK, 1), lambda i: (i, 0)),
    pl.BlockSpec((1, HID), lambda i: (0, 0)),
    pl.BlockSpec((1, HID), lambda i: (0, 0)),
    pl.BlockSpec((HID, HID), lambda i: (0, 0)),
    pl.BlockSpec((1, HID), lambda i: (0, 0)),
    pl.BlockSpec((HID, HID), lambda i: (0, 0)),
    pl.BlockSpec((1, HID), lambda i: (0, 0)),
    pl.BlockSpec((HID, 1), lambda i: (0, 0)),
]

_edge_call = pl.pallas_call(
    _edge_body,
    grid=(E // BLK,),
    in_specs=_edge_in_specs,
    out_specs=[pl.BlockSpec((BLK, HID), lambda i: (i, 0)),
               pl.BlockSpec((BLK, CTW), lambda i: (i, 0))],
    out_shape=[jax.ShapeDtypeStruct((E, HID), F32),
               jax.ShapeDtypeStruct((E, CTW), F32)],
)

_edge_nc_call = pl.pallas_call(
    _edge_body_nc,
    grid=(E // BLK,),
    in_specs=_edge_in_specs[:9],
    out_specs=pl.BlockSpec((BLK, HID), lambda i: (i, 0)),
    out_shape=jax.ShapeDtypeStruct((E, HID), F32),
)


def _node_body(h_ref, c_ref, p1_ref, p2_ref, wn1h_ref, wn1a_ref, bn1_ref,
               wn2_ref, bn2_ref, w1r_ref, w1c_ref, b1_ref,
               h_o, c_o, hr_o, hc_o):
    agg = (p1_ref[0] + p1_ref[1])[:N]
    t16 = (p2_ref[0] + p2_ref[1])[:N]
    cnt = jnp.clip(t16[:, 3:4], 1.0, None)
    lane = lax.broadcasted_iota(jnp.int32, (1, CTW), 1)
    c_o[...] = c_ref[...] + jnp.where(lane < 3, t16, 0.0) / cnt
    h = h_ref[...]
    z = _dot(h, wn1h_ref[...]) + _dot(agg, wn1a_ref[...]) + bn1_ref[...]
    hn = h + _dot(_silu(z), wn2_ref[...]) + bn2_ref[...]
    h_o[...] = hn
    hr_o[...] = _dot(hn, w1r_ref[...]) + b1_ref[...]
    hc_o[...] = _dot(hn, w1c_ref[...])


_node_call = pl.pallas_call(
    _node_body,
    out_shape=[jax.ShapeDtypeStruct((N, HID), F32),
               jax.ShapeDtypeStruct((N, CTW), F32),
               jax.ShapeDtypeStruct((N, HID), F32),
               jax.ShapeDtypeStruct((N, HID), F32)],
)


def _node_final_body(h_ref, p1_ref, batch_ref, wn1h_ref, wn1a_ref, bn1_ref,
                     wn2_ref, bn2_ref, wout_ref, bout_ref, wfc_ref, bfc_ref,
                     o_ref, hscr):
    agg = (p1_ref[0] + p1_ref[1])[:N]
    h = h_ref[...]
    z = _dot(h, wn1h_ref[...]) + _dot(agg, wn1a_ref[...]) + bn1_ref[...]
    hn = h + _dot(_silu(z), wn2_ref[...]) + bn2_ref[...]
    hscr[...] = _dot(hn, wout_ref[...]) + bout_ref[...]
    b = batch_ref[...]
    rowid = lax.broadcasted_iota(jnp.int32, (NG, 1), 0)
    neg = jnp.full((NG, HID), -jnp.inf, F32)

    def body(g, acc):
        mg = jnp.where(b == g, hscr[...], -jnp.inf)
        rmax = jnp.max(mg, axis=0, keepdims=True)
        return jnp.where(rowid == g, rmax, acc)

    gmat = lax.fori_loop(0, NG, body, neg)
    o_ref[...] = _dot(gmat, wfc_ref[...]) + bfc_ref[...]


_node_final_call = pl.pallas_call(
    _node_final_body,
    out_shape=jax.ShapeDtypeStruct((NG, HID), F32),
    scratch_shapes=[pltpu.VMEM((N, HID), F32)],
)


_sc_mesh = plsc.VectorSubcoreMesh(core_axis_name="c", subcore_axis_name="s")


def _gather_body(hr_hbm, hc_hbm, cp_hbm, row_hbm, col_hbm,
                 ohr, ohc, ocr, occ,
                 ridx0, cidx0, bhr0, bhc0, bcr0, bcc0,
                 ridx1, cidx1, bhr1, bhc1, bcr1, bcc1,
                 isem0, gsem0, wsem0, isem1, gsem1, wsem1):
    cid = lax.axis_index("c")
    sid = lax.axis_index("s")
    wid = sid * NC + cid
    tile_base = wid * EDGES_PER_TILE
    RIDX = (ridx0, ridx1)
    CIDX = (cidx0, cidx1)
    BHR = (bhr0, bhr1)
    BHC = (bhc0, bhc1)
    BCR = (bcr0, bcr1)
    BCC = (bcc0, bcc1)
    ISEM = (isem0, isem1)
    GSEM = (gsem0, gsem1)
    WSEM = (wsem0, wsem1)

    def issue_idx(b, ch):
        base = tile_base + ch * CH
        pltpu.async_copy(row_hbm.at[pl.ds(base, CH)], RIDX[b], ISEM[b])
        pltpu.async_copy(col_hbm.at[pl.ds(base, CH)], CIDX[b], ISEM[b])

    def wait_idx(b):
        pltpu.make_async_copy(row_hbm.at[pl.ds(tile_base, CH)], RIDX[b],
                              ISEM[b]).wait()
        pltpu.make_async_copy(col_hbm.at[pl.ds(tile_base, CH)], CIDX[b],
                              ISEM[b]).wait()

    def issue_g(b):
        pltpu.async_copy(hr_hbm.at[RIDX[b]], BHR[b], GSEM[b])
        pltpu.async_copy(hc_hbm.at[CIDX[b]], BHC[b], GSEM[b])
        pltpu.async_copy(cp_hbm.at[RIDX[b]], BCR[b], GSEM[b])
        pltpu.async_copy(cp_hbm.at[CIDX[b]], BCC[b], GSEM[b])

    def wait_g(b):
        pltpu.make_async_copy(hr_hbm.at[RIDX[b]], BHR[b], GSEM[b]).wait()
        pltpu.make_async_copy(hc_hbm.at[CIDX[b]], BHC[b], GSEM[b]).wait()
        pltpu.make_async_copy(cp_hbm.at[RIDX[b]], BCR[b], GSEM[b]).wait()
        pltpu.make_async_copy(cp_hbm.at[CIDX[b]], BCC[b], GSEM[b]).wait()

    def issue_wb(b, ch):
        base = tile_base + ch * CH
        pltpu.async_copy(BHR[b], ohr.at[pl.ds(base, CH)], WSEM[b])
        pltpu.async_copy(BHC[b], ohc.at[pl.ds(base, CH)], WSEM[b])
        pltpu.async_copy(BCR[b], ocr.at[pl.ds(base, CH)], WSEM[b])
        pltpu.async_copy(BCC[b], occ.at[pl.ds(base, CH)], WSEM[b])

    def wait_wb(b):
        pltpu.make_async_copy(BHR[b], ohr.at[pl.ds(tile_base, CH)],
                              WSEM[b]).wait()
        pltpu.make_async_copy(BHC[b], ohc.at[pl.ds(tile_base, CH)],
                              WSEM[b]).wait()
        pltpu.make_async_copy(BCR[b], ocr.at[pl.ds(tile_base, CH)],
                              WSEM[b]).wait()
        pltpu.make_async_copy(BCC[b], occ.at[pl.ds(tile_base, CH)],
                              WSEM[b]).wait()

    issue_idx(0, 0)
    wait_idx(0)
    issue_g(0)
    issue_idx(1, 1)

    @pl.loop(0, (CHUNKS_PER_TILE - 1) // 2)
    def _(p):
        a = 2 * p
        bch = 2 * p + 1
        nxt0 = 2 * p + 2
        nxt1 = jnp.minimum(2 * p + 3, CHUNKS_PER_TILE - 1)
        wait_g(0)
        issue_wb(0, a)
        wait_idx(1)
        issue_g(1)
        issue_idx(0, nxt0)
        wait_g(1)
        issue_wb(1, bch)
        issue_idx(1, nxt1)
        wait_wb(0)
        wait_idx(0)
        issue_g(0)
        wait_wb(1)

    wait_g(0)
    issue_wb(0, CHUNKS_PER_TILE - 1)
    wait_wb(0)
    wait_idx(1)


def _sc_gather(hr, hc, cp, row, col):
    bufs = [pltpu.VMEM((CH,), jnp.int32),
            pltpu.VMEM((CH,), jnp.int32),
            pltpu.VMEM((CH, HID), F32),
            pltpu.VMEM((CH, HID), F32),
            pltpu.VMEM((CH, CTW), F32),
            pltpu.VMEM((CH, CTW), F32)]
    f = pl.kernel(
        _gather_body,
        out_type=[jax.ShapeDtypeStruct((E, HID), F32),
                  jax.ShapeDtypeStruct((E, HID), F32),
                  jax.ShapeDtypeStruct((E, CTW), F32),
                  jax.ShapeDtypeStruct((E, CTW), F32)],
        mesh=_sc_mesh,
        scratch_types=bufs + bufs + [pltpu.SemaphoreType.DMA] * 6,
    )
    return f(hr, hc, cp, row, col)


def _scatter1_body(m_hbm, row_hbm, z1_hbm, o1_hbm,
                   idx0, pay0, idx1, pay1, acc1,
                   lsem0, ssem0, lsem1, ssem1):
    cid = lax.axis_index("c")
    sid = lax.axis_index("s")
    wid = sid * NC + cid
    tile_base = wid * EDGES_PER_TILE
    r0 = sid * ROWS_PER_SUB
    IDX = (idx0, idx1)
    PAY = (pay0, pay1)
    LSEM = (lsem0, lsem1)
    SSEM = (ssem0, ssem1)

    def issue_load(b, ch):
        base = tile_base + ch * CH
        pltpu.async_copy(row_hbm.at[pl.ds(base, CH)], IDX[b], LSEM[b])
        pltpu.async_copy(m_hbm.at[pl.ds(base, CH)], PAY[b], LSEM[b])

    def wait_load(b):
        pltpu.make_async_copy(row_hbm.at[pl.ds(tile_base, CH)], IDX[b],
                              LSEM[b]).wait()
        pltpu.make_async_copy(m_hbm.at[pl.ds(tile_base, CH)], PAY[b],
                              LSEM[b]).wait()

    def issue_sc(b):
        pltpu.async_copy(PAY[b], acc1.at[IDX[b]], SSEM[b], add=True)

    def wait_sc(b):
        pltpu.make_async_copy(PAY[b], acc1.at[IDX[b]], SSEM[b]).wait()

    pltpu.sync_copy(z1_hbm.at[pl.ds(r0, ROWS_PER_SUB)],
                    acc1.at[pl.ds(r0, ROWS_PER_SUB)])
    plsc.subcore_barrier()

    issue_load(0, 0)
    issue_load(1, 1)

    @pl.loop(0, (CHUNKS_PER_TILE - 1) // 2)
    def _(p):
        nxt0 = 2 * p + 2
        nxt1 = jnp.minimum(2 * p + 3, CHUNKS_PER_TILE - 1)
        wait_load(0)
        issue_sc(0)
        wait_load(1)
        issue_sc(1)
        wait_sc(0)
        issue_load(0, nxt0)
        wait_sc(1)
        issue_load(1, nxt1)

    wait_load(0)
    issue_sc(0)
    wait_load(1)
    wait_sc(0)

    plsc.subcore_barrier()
    pltpu.sync_copy(acc1.at[pl.ds(r0, ROWS_PER_SUB)],
                    o1_hbm.at[cid, pl.ds(r0, ROWS_PER_SUB)])


def _sc_scatter1(m_e, row, z1):
    f = pl.kernel(
        _scatter1_body,
        out_type=jax.ShapeDtypeStruct((NC, NPAD, HID), F32),
        mesh=_sc_mesh,
        scratch_types=[pltpu.VMEM((CH,), jnp.int32),
                       pltpu.VMEM((CH, HID), F32),
                       pltpu.VMEM((CH,), jnp.int32),
                       pltpu.VMEM((CH, HID), F32),
                       pltpu.VMEM_SHARED((NPAD, HID), F32),
                       pltpu.SemaphoreType.DMA,
                       pltpu.SemaphoreType.DMA,
                       pltpu.SemaphoreType.DMA,
                       pltpu.SemaphoreType.DMA],
    )
    return f(m_e, row, z1)


def kernel(x, edge_index, coord, edge_attr, batch, ids, id_table, W_in, b_in,
           We1, be1, We2, be2, Wn1, bn1, Wn2, bn2, Wc1, bc1, Wc2,
           W_out, b_out, W_fc, b_fc):
    row = edge_index[0].astype(jnp.int32)
    col = edge_index[1].astype(jnp.int32)
    cpad = jnp.pad(coord.astype(F32), ((0, 0), (0, CTW - 3)))
    ids2 = ids.reshape(N, 1).astype(jnp.int32)
    batch2 = batch.reshape(N, 1).astype(jnp.int32)
    z1 = jnp.zeros((NPAD, HID), F32)

    h, hr, hc = _pre_call(x, ids2, id_table, W_in, b_in.reshape(1, HID),
                          We1[0, :HID], We1[0, HID:2 * HID],
                          be1[0].reshape(1, HID))
    c = cpad
    out = None
    for i in range(3):
        hr_g, hc_g, cr_g, cc_g = _sc_gather(hr, hc, c, row, col)
        wrad = We1[i, 2 * HID:2 * HID + 1]
        wea = We1[i, 2 * HID + 1:2 * HID + 2]
        if i < 2:
            m_e, tr_e = _edge_call(hr_g, hc_g, cr_g, cc_g, edge_attr, wrad,
                                   wea, We2[i], be2[i].reshape(1, HID),
                                   Wc1[i], bc1[i].reshape(1, HID), Wc2[i])
            p1 = _sc_scatter1(m_e, row, z1)
            p2 = _sc_scatter1(tr_e, row, z1)
            h, c, hr, hc = _node_call(h, c, p1, p2, Wn1[i, :HID],
                                      Wn1[i, HID:], bn1[i].reshape(1, HID),
                                      Wn2[i], bn2[i].reshape(1, HID),
                                      We1[i + 1, :HID],
                                      We1[i + 1, HID:2 * HID],
                                      be1[i + 1].reshape(1, HID))
        else:
            m_e = _edge_nc_call(hr_g, hc_g, cr_g, cc_g, edge_attr, wrad, wea,
                                We2[i], be2[i].reshape(1, HID))
            p1 = _sc_scatter1(m_e, row, z1)
            out = _node_final_call(h, p1, batch2, Wn1[i, :HID], Wn1[i, HID:],
                                   bn1[i].reshape(1, HID), Wn2[i],
                                   bn2[i].reshape(1, HID), W_out,
                                   b_out.reshape(1, HID), W_fc,
                                   b_fc.reshape(1, HID))
    return out
